# baseline (speedup 1.0000x reference)
import functools

import jax
import jax.numpy as jnp
from jax import lax
from jax.experimental import pallas as pl
from jax.experimental.pallas import tpu as pltpu
from jax.experimental.pallas import tpu_sc as plsc

B = 16384
NF = 26
V = 100000
D = 10
DP = 16
NUM = 13
H1 = 128
H2 = 64
EW = NF * DP

NC = 2
NS = 16
NW = NC * NS
ROWS = B * NF
RPW = ROWS // NW
G = 128
NG = RPW // G
CH = 26
NCH = NG // CH
CROWS = CH * G


def _gather_body(tab_hbm, idx_hbm, out_hbm, idx_v, buf0, buf1,
                 gsem0, gsem1, osem0, osem1):
    wid = lax.axis_index("s") * NC + lax.axis_index("c")
    pltpu.sync_copy(idx_hbm.at[wid], idx_v)
    bufs = (buf0, buf1)
    gsems = (gsem0, gsem1)
    osems = (osem0, osem1)
    cps = [None, None]
    ocs = [None, None]

    def fire(ch):
        b = ch & 1
        cps[b] = [
            pltpu.async_copy(
                tab_hbm.at[idx_v.at[ch * CH + j]],
                bufs[b].at[pl.ds(j * G, G)],
                gsems[b])
            for j in range(CH)
        ]

    fire(0)
    for ch in range(NCH):
        b = ch & 1
        if ch + 1 < NCH:
            nb = (ch + 1) & 1
            if ocs[nb] is not None:
                ocs[nb].wait()
            fire(ch + 1)
        for cp in cps[b]:
            cp.wait()
        oc = pltpu.make_async_copy(
            bufs[b], out_hbm.at[pl.ds(wid * RPW + ch * CROWS, CROWS)],
            osems[b])
        oc.start()
        ocs[b] = oc
    ocs[(NCH - 2) & 1].wait()
    ocs[(NCH - 1) & 1].wait()


@functools.lru_cache(maxsize=1)
def _make_gather():
    return pl.kernel(
        _gather_body,
        out_type=jax.ShapeDtypeStruct((ROWS, DP), jnp.float32),
        mesh=plsc.VectorSubcoreMesh(
            core_axis_name="c", subcore_axis_name="s",
            num_cores=NC, num_subcores=NS),
        scratch_types=[
            pltpu.VMEM((NG, G), jnp.int32),
            pltpu.VMEM((CROWS, DP), jnp.float32),
            pltpu.VMEM((CROWS, DP), jnp.float32),
            pltpu.SemaphoreType.DMA,
            pltpu.SemaphoreType.DMA,
            pltpu.SemaphoreType.DMA,
            pltpu.SemaphoreType.DMA,
        ],
        compiler_params=pltpu.CompilerParams(use_tc_tiling_on_sc=False),
    )


VB = 1024
NVB = -(-V // VB)
VP = NVB * VB


def _repack_body(tp_ref, out_ref):
    x = tp_ref[...]
    i0 = lax.broadcasted_iota(jnp.int32, (D, DP), 0)
    i1 = lax.broadcasted_iota(jnp.int32, (D, DP), 1)
    eye = (i0 == i1).astype(jnp.float32)
    out_ref[...] = lax.dot_general(
        x, eye, (((0,), (0,)), ((), ())), preferred_element_type=jnp.float32)


_repack = pl.pallas_call(
    _repack_body,
    grid=(NVB,),
    in_specs=[pl.BlockSpec((D, NF, VB), lambda v: (0, 0, v))],
    out_specs=pl.BlockSpec((NF, VB, DP), lambda v: (0, v, 0)),
    out_shape=jax.ShapeDtypeStruct((NF, VP, DP), jnp.float32),
)


BLK = 2048
NBLK = B // BLK
_EPS = 1e-5


def _fm_from_emb(emb):
    i0 = lax.broadcasted_iota(jnp.int32, (EW, D), 0)
    i1 = lax.broadcasted_iota(jnp.int32, (EW, D), 1)
    sel = (i0 % DP == i1).astype(jnp.float32)
    sum_emb = jnp.dot(emb, sel, preferred_element_type=jnp.float32)
    sq_sum = jnp.dot(emb * emb, sel, preferred_element_type=jnp.float32)
    return 0.5 * (sum_emb * sum_emb - sq_sum)


def _mlp_body(num_ref, emb_ref,
              g0n_ref, b0n_ref, g0e_ref, b0e_ref, g0f_ref, b0f_ref,
              w1n_ref, w1e_ref, w1f_ref, b1_ref, g1_ref, bb1_ref,
              w2_ref, b2_ref, g2_ref, bb2_ref, w3_ref, b3_ref,
              out_ref,
              h1_s, h2_s,
              sn_s, qn_s, se_s, qe_s, sf_s, qf_s, s1_s, q1_s, s2_s, q2_s):
    f32 = jnp.float32
    ph = pl.program_id(0)
    j = pl.program_id(1)
    rows = pl.ds(j * BLK, BLK)

    def colstats(p):
        return (jnp.sum(p, axis=0, keepdims=True),
                jnp.sum(p * p, axis=0, keepdims=True))

    def scale_shift(s_ref, q_ref, g, b):
        m = s_ref[...] * (1.0 / B)
        v = q_ref[...] * (1.0 / B) - m * m
        sc = g / jnp.sqrt(v + _EPS)
        return sc, b - m * sc

    @pl.when(ph == 0)
    def _phase0():
        @pl.when(j == 0)
        def _zero():
            for r in (sn_s, qn_s, se_s, qe_s, sf_s, qf_s, s1_s, q1_s, s2_s, q2_s):
                r[...] = jnp.zeros_like(r)
        num = num_ref[...]
        emb = emb_ref[...]
        fm = _fm_from_emb(emb)
        s, q = colstats(num); sn_s[...] += s; qn_s[...] += q
        s, q = colstats(emb); se_s[...] += s; qe_s[...] += q
        s, q = colstats(fm); sf_s[...] += s; qf_s[...] += q

    @pl.when(ph == 1)
    def _phase1():
        num = num_ref[...]
        emb = emb_ref[...]
        fm = _fm_from_emb(emb)
        scn, shn = scale_shift(sn_s, qn_s, g0n_ref[...], b0n_ref[...])
        sce, she = scale_shift(se_s, qe_s, g0e_ref[...], b0e_ref[...])
        scf, shf = scale_shift(sf_s, qf_s, g0f_ref[...], b0f_ref[...])
        h1 = (jnp.dot(num * scn + shn, w1n_ref[...], preferred_element_type=f32)
              + jnp.dot(emb * sce + she, w1e_ref[...], preferred_element_type=f32)
              + jnp.dot(fm * scf + shf, w1f_ref[...], preferred_element_type=f32)
              + b1_ref[...])
        s, q = colstats(h1); s1_s[...] += s; q1_s[...] += q
        h1_s[rows, :] = h1

    @pl.when(ph == 2)
    def _phase2():
        h1 = h1_s[rows, :]
        sc, sh = scale_shift(s1_s, q1_s, g1_ref[...], bb1_ref[...])
        h1 = jnp.maximum(h1 * sc + sh, 0.0)
        h2 = jnp.dot(h1, w2_ref[...], preferred_element_type=f32) + b2_ref[...]
        s, q = colstats(h2); s2_s[...] += s; q2_s[...] += q
        h2_s[rows, :] = h2

    @pl.when(ph == 3)
    def _phase3():
        h2 = h2_s[rows, :]
        sc, sh = scale_shift(s2_s, q2_s, g2_ref[...], bb2_ref[...])
        h2 = jnp.maximum(h2 * sc + sh, 0.0)
        out_ref[...] = jnp.dot(h2, w3_ref[...], preferred_element_type=f32) + b3_ref[...]


def _big_spec(ncols):
    return pl.BlockSpec(
        (BLK, ncols), lambda ph, j: (jnp.where(ph <= 1, j, 0), 0))


def _w_spec(shape):
    return pl.BlockSpec(shape, lambda ph, j: (0,) * len(shape))


_mlp = pl.pallas_call(
    _mlp_body,
    grid=(4, NBLK),
    in_specs=[
        _big_spec(NUM), _big_spec(EW),
        _w_spec((1, NUM)), _w_spec((1, NUM)),
        _w_spec((1, EW)), _w_spec((1, EW)),
        _w_spec((1, D)), _w_spec((1, D)),
        _w_spec((NUM, H1)), _w_spec((EW, H1)), _w_spec((D, H1)),
        _w_spec((1, H1)), _w_spec((1, H1)), _w_spec((1, H1)),
        _w_spec((H1, H2)), _w_spec((1, H2)), _w_spec((1, H2)), _w_spec((1, H2)),
        _w_spec((H2, 1)), _w_spec((1, 1)),
    ],
    out_specs=pl.BlockSpec((BLK, 1), lambda ph, j: (j, 0)),
    out_shape=jax.ShapeDtypeStruct((B, 1), jnp.float32),
    scratch_shapes=[
        pltpu.VMEM((B, H1), jnp.float32),
        pltpu.VMEM((B, H2), jnp.float32),
        pltpu.VMEM((1, NUM), jnp.float32), pltpu.VMEM((1, NUM), jnp.float32),
        pltpu.VMEM((1, EW), jnp.float32), pltpu.VMEM((1, EW), jnp.float32),
        pltpu.VMEM((1, D), jnp.float32), pltpu.VMEM((1, D), jnp.float32),
        pltpu.VMEM((1, H1), jnp.float32), pltpu.VMEM((1, H1), jnp.float32),
        pltpu.VMEM((1, H2), jnp.float32), pltpu.VMEM((1, H2), jnp.float32),
    ],
)


def _pad_fd(a):
    a = a.reshape(a.shape[:-1] + (NF, D))
    pad = [(0, 0)] * (a.ndim - 1) + [(0, DP - D)]
    return jnp.pad(a, pad).reshape(a.shape[:-2] + (NF * DP,))


def kernel(numerical, categorical, tables, bn0_g, bn0_b, w1, b1, bn1_g,
           bn1_b, w2, b2, bn2_g, bn2_b, w3, b3):
    tab_pad = _repack(jnp.transpose(tables, (2, 0, 1))).reshape(NF * VP, DP)
    offs = (jnp.arange(NF, dtype=jnp.int32) * VP)[None, :]
    flat_idx = (categorical.astype(jnp.int32) + offs).reshape(NW, NG, G)
    emb = _make_gather()(tab_pad, flat_idx).reshape(B, EW)

    e0, e1 = NUM, NUM + NF * D
    out = _mlp(
        numerical, emb,
        bn0_g[None, :e0], bn0_b[None, :e0],
        _pad_fd(bn0_g[e0:e1])[None, :], _pad_fd(bn0_b[e0:e1])[None, :],
        bn0_g[None, e1:], bn0_b[None, e1:],
        w1[:, :e0].T, _pad_fd(w1[:, e0:e1]).T, w1[:, e1:].T,
        b1[None, :], bn1_g[None, :], bn1_b[None, :],
        w2.T, b2[None, :], bn2_g[None, :], bn2_b[None, :],
        w3.T, b3[None, :],
    )
    return out


# speedup vs baseline: 4.7527x; 4.7527x over previous
import functools

import jax
import jax.numpy as jnp
from jax import lax
from jax.experimental import pallas as pl
from jax.experimental.pallas import tpu as pltpu
from jax.experimental.pallas import tpu_sc as plsc

B = 16384
NF = 26
V = 100000
D = 10
DP = 16
NUM = 13
H1 = 128
H2 = 64
EW = NF * DP

NC = 2
NS = 16
NW = NC * NS
ROWS = B * NF
RPW = ROWS // NW
G = 128
NG = RPW // G
CH = 26
NCH = NG // CH
CROWS = CH * G


def _gather_body(tab_hbm, idx_hbm, out_hbm, idx_v, buf0, buf1,
                 gsem0, gsem1, osem0, osem1):
    wid = lax.axis_index("s") * NC + lax.axis_index("c")
    pltpu.sync_copy(idx_hbm.at[wid], idx_v)
    bufs = (buf0, buf1)
    gsems = (gsem0, gsem1)
    osems = (osem0, osem1)
    cps = [None, None]
    ocs = [None, None]

    def fire(ch):
        b = ch & 1
        cps[b] = [
            pltpu.async_copy(
                tab_hbm.at[idx_v.at[ch * CH + j]],
                bufs[b].at[pl.ds(j * G, G)],
                gsems[b])
            for j in range(CH)
        ]

    fire(0)
    for ch in range(NCH):
        b = ch & 1
        if ch + 1 < NCH:
            nb = (ch + 1) & 1
            if ocs[nb] is not None:
                ocs[nb].wait()
            fire(ch + 1)
        for cp in cps[b]:
            cp.wait()
        oc = pltpu.make_async_copy(
            bufs[b], out_hbm.at[pl.ds(wid * RPW + ch * CROWS, CROWS)],
            osems[b])
        oc.start()
        ocs[b] = oc
    ocs[(NCH - 2) & 1].wait()
    ocs[(NCH - 1) & 1].wait()


@functools.lru_cache(maxsize=1)
def _make_gather():
    return pl.kernel(
        _gather_body,
        out_type=jax.ShapeDtypeStruct((ROWS, DP), jnp.float32),
        mesh=plsc.VectorSubcoreMesh(
            core_axis_name="c", subcore_axis_name="s",
            num_cores=NC, num_subcores=NS),
        scratch_types=[
            pltpu.VMEM((NG, G), jnp.int32),
            pltpu.VMEM((CROWS, DP), jnp.float32),
            pltpu.VMEM((CROWS, DP), jnp.float32),
            pltpu.SemaphoreType.DMA,
            pltpu.SemaphoreType.DMA,
            pltpu.SemaphoreType.DMA,
            pltpu.SemaphoreType.DMA,
        ],
        compiler_params=pltpu.CompilerParams(use_tc_tiling_on_sc=False),
    )


VB = 1024
NVB = -(-V // VB)
VP = NVB * VB
QF = VP // 8


def _repack_body(tp_ref, out_ref):
    x = tp_ref[...]
    xc = jnp.concatenate(
        [x[:, :, 128 * s:128 * (s + 1)] for s in range(8)], axis=0)
    i0 = lax.broadcasted_iota(jnp.int32, (8 * D, 128), 0)
    i1 = lax.broadcasted_iota(jnp.int32, (8 * D, 128), 1)
    place = (i1 == (i0 // D) * DP + (i0 % D)).astype(jnp.float32)
    out_ref[...] = lax.dot_general(
        xc, place, (((0,), (0,)), ((), ())), preferred_element_type=jnp.float32)


_repack = pl.pallas_call(
    _repack_body,
    grid=(NVB,),
    in_specs=[pl.BlockSpec((D, NF, VB), lambda v: (0, 0, v))],
    out_specs=pl.BlockSpec((NF, 128, 128), lambda v: (0, v, 0)),
    out_shape=jax.ShapeDtypeStruct((NF, QF, 128), jnp.float32),
)


BLK = 2048
NBLK = B // BLK
_EPS = 1e-5


def _fm_from_emb(emb):
    i0 = lax.broadcasted_iota(jnp.int32, (EW, D), 0)
    i1 = lax.broadcasted_iota(jnp.int32, (EW, D), 1)
    sel = (i0 % DP == i1).astype(jnp.float32)
    sum_emb = jnp.dot(emb, sel, preferred_element_type=jnp.float32)
    sq_sum = jnp.dot(emb * emb, sel, preferred_element_type=jnp.float32)
    return 0.5 * (sum_emb * sum_emb - sq_sum)


def _mlp_body(num_ref, emb_ref,
              g0n_ref, b0n_ref, g0e_ref, b0e_ref, g0f_ref, b0f_ref,
              w1n_ref, w1e_ref, w1f_ref, b1_ref, g1_ref, bb1_ref,
              w2_ref, b2_ref, g2_ref, bb2_ref, w3_ref, b3_ref,
              out_ref,
              h1_s, h2_s,
              sn_s, qn_s, se_s, qe_s, sf_s, qf_s, s1_s, q1_s, s2_s, q2_s):
    f32 = jnp.float32
    ph = pl.program_id(0)
    j = pl.program_id(1)
    rows = pl.ds(j * BLK, BLK)

    def colstats(p):
        return (jnp.sum(p, axis=0, keepdims=True),
                jnp.sum(p * p, axis=0, keepdims=True))

    def scale_shift(s_ref, q_ref, g, b):
        m = s_ref[...] * (1.0 / B)
        v = q_ref[...] * (1.0 / B) - m * m
        sc = g / jnp.sqrt(v + _EPS)
        return sc, b - m * sc

    @pl.when(ph == 0)
    def _phase0():
        @pl.when(j == 0)
        def _zero():
            for r in (sn_s, qn_s, se_s, qe_s, sf_s, qf_s, s1_s, q1_s, s2_s, q2_s):
                r[...] = jnp.zeros_like(r)
        num = num_ref[...]
        emb = emb_ref[...]
        fm = _fm_from_emb(emb)
        s, q = colstats(num); sn_s[...] += s; qn_s[...] += q
        s, q = colstats(emb); se_s[...] += s; qe_s[...] += q
        s, q = colstats(fm); sf_s[...] += s; qf_s[...] += q

    @pl.when(ph == 1)
    def _phase1():
        num = num_ref[...]
        emb = emb_ref[...]
        fm = _fm_from_emb(emb)
        scn, shn = scale_shift(sn_s, qn_s, g0n_ref[...], b0n_ref[...])
        sce, she = scale_shift(se_s, qe_s, g0e_ref[...], b0e_ref[...])
        scf, shf = scale_shift(sf_s, qf_s, g0f_ref[...], b0f_ref[...])
        h1 = (jnp.dot(num * scn + shn, w1n_ref[...], preferred_element_type=f32)
              + jnp.dot(emb * sce + she, w1e_ref[...], preferred_element_type=f32)
              + jnp.dot(fm * scf + shf, w1f_ref[...], preferred_element_type=f32)
              + b1_ref[...])
        s, q = colstats(h1); s1_s[...] += s; q1_s[...] += q
        h1_s[rows, :] = h1

    @pl.when(ph == 2)
    def _phase2():
        h1 = h1_s[rows, :]
        sc, sh = scale_shift(s1_s, q1_s, g1_ref[...], bb1_ref[...])
        h1 = jnp.maximum(h1 * sc + sh, 0.0)
        h2 = jnp.dot(h1, w2_ref[...], preferred_element_type=f32) + b2_ref[...]
        s, q = colstats(h2); s2_s[...] += s; q2_s[...] += q
        h2_s[rows, :] = h2

    @pl.when(ph == 3)
    def _phase3():
        h2 = h2_s[rows, :]
        sc, sh = scale_shift(s2_s, q2_s, g2_ref[...], bb2_ref[...])
        h2 = jnp.maximum(h2 * sc + sh, 0.0)
        out_ref[...] = jnp.dot(h2, w3_ref[...], preferred_element_type=f32) + b3_ref[...]


def _big_spec(ncols):
    return pl.BlockSpec(
        (BLK, ncols), lambda ph, j: (jnp.where(ph <= 1, j, 0), 0))


def _w_spec(shape):
    return pl.BlockSpec(shape, lambda ph, j: (0,) * len(shape))


_mlp = pl.pallas_call(
    _mlp_body,
    grid=(4, NBLK),
    in_specs=[
        _big_spec(NUM), _big_spec(EW),
        _w_spec((1, NUM)), _w_spec((1, NUM)),
        _w_spec((1, EW)), _w_spec((1, EW)),
        _w_spec((1, D)), _w_spec((1, D)),
        _w_spec((NUM, H1)), _w_spec((EW, H1)), _w_spec((D, H1)),
        _w_spec((1, H1)), _w_spec((1, H1)), _w_spec((1, H1)),
        _w_spec((H1, H2)), _w_spec((1, H2)), _w_spec((1, H2)), _w_spec((1, H2)),
        _w_spec((H2, 1)), _w_spec((1, 1)),
    ],
    out_specs=pl.BlockSpec((BLK, 1), lambda ph, j: (j, 0)),
    out_shape=jax.ShapeDtypeStruct((B, 1), jnp.float32),
    scratch_shapes=[
        pltpu.VMEM((B, H1), jnp.float32),
        pltpu.VMEM((B, H2), jnp.float32),
        pltpu.VMEM((1, NUM), jnp.float32), pltpu.VMEM((1, NUM), jnp.float32),
        pltpu.VMEM((1, EW), jnp.float32), pltpu.VMEM((1, EW), jnp.float32),
        pltpu.VMEM((1, D), jnp.float32), pltpu.VMEM((1, D), jnp.float32),
        pltpu.VMEM((1, H1), jnp.float32), pltpu.VMEM((1, H1), jnp.float32),
        pltpu.VMEM((1, H2), jnp.float32), pltpu.VMEM((1, H2), jnp.float32),
    ],
)


def _pad_fd(a):
    a = a.reshape(a.shape[:-1] + (NF, D))
    pad = [(0, 0)] * (a.ndim - 1) + [(0, DP - D)]
    return jnp.pad(a, pad).reshape(a.shape[:-2] + (NF * DP,))


def kernel(numerical, categorical, tables, bn0_g, bn0_b, w1, b1, bn1_g,
           bn1_b, w2, b2, bn2_g, bn2_b, w3, b3):
    tab_pad = _repack(jnp.transpose(tables, (2, 0, 1))).reshape(NF * VP, DP)
    cat = categorical.astype(jnp.int32)
    offs = (jnp.arange(NF, dtype=jnp.int32) * VP)[None, :]
    flat_idx = (offs + (cat // VB) * VB + (cat % 128) * 8
                + (cat % VB) // 128).reshape(NW, NG, G)
    emb = _make_gather()(tab_pad, flat_idx).reshape(B, EW)

    e0, e1 = NUM, NUM + NF * D
    out = _mlp(
        numerical, emb,
        bn0_g[None, :e0], bn0_b[None, :e0],
        _pad_fd(bn0_g[e0:e1])[None, :], _pad_fd(bn0_b[e0:e1])[None, :],
        bn0_g[None, e1:], bn0_b[None, e1:],
        w1[:, :e0].T, _pad_fd(w1[:, e0:e1]).T, w1[:, e1:].T,
        b1[None, :], bn1_g[None, :], bn1_b[None, :],
        w2.T, b2[None, :], bn2_g[None, :], bn2_b[None, :],
        w3.T, b3[None, :],
    )
    return out


# speedup vs baseline: 5.4054x; 1.1373x over previous
import functools

import jax
import jax.numpy as jnp
from jax import lax
from jax.experimental import pallas as pl
from jax.experimental.pallas import tpu as pltpu
from jax.experimental.pallas import tpu_sc as plsc

B = 16384
NF = 26
V = 100000
D = 10
DP = 16
NUM = 13
H1 = 128
H2 = 64
EW = NF * DP

NC = 2
NS = 16
NW = NC * NS
ROWS = B * NF
RPW = ROWS // NW
G = 128
NG = RPW // G
CH = 26
NCH = NG // CH
CROWS = CH * G


def _gather_body(tab_hbm, idx_hbm, out_hbm, idx_v, buf0, buf1,
                 gsem0, gsem1, osem0, osem1):
    wid = lax.axis_index("s") * NC + lax.axis_index("c")
    pltpu.sync_copy(idx_hbm.at[wid], idx_v)
    bufs = (buf0, buf1)
    gsems = (gsem0, gsem1)
    osems = (osem0, osem1)
    cps = [None, None]
    ocs = [None, None]

    def fire(ch):
        b = ch & 1
        cps[b] = [
            pltpu.async_copy(
                tab_hbm.at[idx_v.at[ch * CH + j]],
                bufs[b].at[pl.ds(j * G, G)],
                gsems[b])
            for j in range(CH)
        ]

    fire(0)
    for ch in range(NCH):
        b = ch & 1
        if ch + 1 < NCH:
            nb = (ch + 1) & 1
            if ocs[nb] is not None:
                ocs[nb].wait()
            fire(ch + 1)
        for cp in cps[b]:
            cp.wait()
        oc = pltpu.make_async_copy(
            bufs[b], out_hbm.at[pl.ds(wid * RPW + ch * CROWS, CROWS)],
            osems[b])
        oc.start()
        ocs[b] = oc
    ocs[(NCH - 2) & 1].wait()
    ocs[(NCH - 1) & 1].wait()


@functools.lru_cache(maxsize=1)
def _make_gather():
    return pl.kernel(
        _gather_body,
        out_type=jax.ShapeDtypeStruct((ROWS, DP), jnp.float32),
        mesh=plsc.VectorSubcoreMesh(
            core_axis_name="c", subcore_axis_name="s",
            num_cores=NC, num_subcores=NS),
        scratch_types=[
            pltpu.VMEM((NG, G), jnp.int32),
            pltpu.VMEM((CROWS, DP), jnp.float32),
            pltpu.VMEM((CROWS, DP), jnp.float32),
            pltpu.SemaphoreType.DMA,
            pltpu.SemaphoreType.DMA,
            pltpu.SemaphoreType.DMA,
            pltpu.SemaphoreType.DMA,
        ],
        compiler_params=pltpu.CompilerParams(use_tc_tiling_on_sc=False),
    )


VB = 2048
NVB = -(-V // VB)
VP = NVB * VB
QF = VP // 8
RR = VB // 8


def _repack_body(tp_ref, out_ref):
    x = tp_ref[...]
    xc = jnp.concatenate(
        [x[:, :, RR * s:RR * (s + 1)] for s in range(8)], axis=0)
    i0 = lax.broadcasted_iota(jnp.int32, (8 * D, 128), 0)
    i1 = lax.broadcasted_iota(jnp.int32, (8 * D, 128), 1)
    place = (i1 == (i0 // D) * DP + (i0 % D)).astype(jnp.float32)
    out_ref[...] = lax.dot_general(
        xc, place, (((0,), (0,)), ((), ())), preferred_element_type=jnp.float32)


_repack = pl.pallas_call(
    _repack_body,
    grid=(NVB,),
    in_specs=[pl.BlockSpec((D, NF, VB), lambda v: (0, 0, v))],
    out_specs=pl.BlockSpec((NF, RR, 128), lambda v: (0, v, 0)),
    out_shape=jax.ShapeDtypeStruct((NF, QF, 128), jnp.float32),
)


BLK = 4096
NBLK = B // BLK
_EPS = 1e-5


def _fm_from_emb(emb):
    i0 = lax.broadcasted_iota(jnp.int32, (EW, D), 0)
    i1 = lax.broadcasted_iota(jnp.int32, (EW, D), 1)
    sel = (i0 % DP == i1).astype(jnp.float32)
    sum_emb = jnp.dot(emb, sel, preferred_element_type=jnp.float32)
    sq_sum = jnp.dot(emb * emb, sel, preferred_element_type=jnp.float32)
    return 0.5 * (sum_emb * sum_emb - sq_sum)


def _mlp_body(num_ref, emb_ref,
              g0n_ref, b0n_ref, g0e_ref, b0e_ref, g0f_ref, b0f_ref,
              w1n_ref, w1e_ref, w1f_ref, b1_ref, g1_ref, bb1_ref,
              w2_ref, b2_ref, g2_ref, bb2_ref, w3_ref, b3_ref,
              out_ref,
              h1_s, h2_s,
              sn_s, qn_s, se_s, qe_s, sf_s, qf_s, s1_s, q1_s, s2_s, q2_s):
    f32 = jnp.float32
    ph = pl.program_id(0)
    j = pl.program_id(1)
    rows = pl.ds(j * BLK, BLK)

    def colstats(p):
        return (jnp.sum(p, axis=0, keepdims=True),
                jnp.sum(p * p, axis=0, keepdims=True))

    def scale_shift(s_ref, q_ref, g, b):
        m = s_ref[...] * (1.0 / B)
        v = q_ref[...] * (1.0 / B) - m * m
        sc = g / jnp.sqrt(v + _EPS)
        return sc, b - m * sc

    @pl.when(ph == 0)
    def _phase0():
        @pl.when(j == 0)
        def _zero():
            for r in (sn_s, qn_s, se_s, qe_s, sf_s, qf_s, s1_s, q1_s, s2_s, q2_s):
                r[...] = jnp.zeros_like(r)
        num = num_ref[...]
        emb = emb_ref[...]
        fm = _fm_from_emb(emb)
        s, q = colstats(num); sn_s[...] += s; qn_s[...] += q
        s, q = colstats(emb); se_s[...] += s; qe_s[...] += q
        s, q = colstats(fm); sf_s[...] += s; qf_s[...] += q

    @pl.when(ph == 1)
    def _phase1():
        num = num_ref[...]
        emb = emb_ref[...]
        fm = _fm_from_emb(emb)
        scn, shn = scale_shift(sn_s, qn_s, g0n_ref[...], b0n_ref[...])
        sce, she = scale_shift(se_s, qe_s, g0e_ref[...], b0e_ref[...])
        scf, shf = scale_shift(sf_s, qf_s, g0f_ref[...], b0f_ref[...])
        h1 = (jnp.dot(num * scn + shn, w1n_ref[...], preferred_element_type=f32)
              + jnp.dot(emb * sce + she, w1e_ref[...], preferred_element_type=f32)
              + jnp.dot(fm * scf + shf, w1f_ref[...], preferred_element_type=f32)
              + b1_ref[...])
        s, q = colstats(h1); s1_s[...] += s; q1_s[...] += q
        h1_s[rows, :] = h1

    @pl.when(ph == 2)
    def _phase2():
        h1 = h1_s[rows, :]
        sc, sh = scale_shift(s1_s, q1_s, g1_ref[...], bb1_ref[...])
        h1 = jnp.maximum(h1 * sc + sh, 0.0)
        h2 = jnp.dot(h1, w2_ref[...], preferred_element_type=f32) + b2_ref[...]
        s, q = colstats(h2); s2_s[...] += s; q2_s[...] += q
        h2_s[rows, :] = h2

    @pl.when(ph == 3)
    def _phase3():
        h2 = h2_s[rows, :]
        sc, sh = scale_shift(s2_s, q2_s, g2_ref[...], bb2_ref[...])
        h2 = jnp.maximum(h2 * sc + sh, 0.0)
        out_ref[...] = jnp.dot(h2, w3_ref[...], preferred_element_type=f32) + b3_ref[...]


def _big_spec(ncols):
    return pl.BlockSpec(
        (BLK, ncols), lambda ph, j: (jnp.where(ph <= 1, j, 0), 0))


def _w_spec(shape):
    return pl.BlockSpec(shape, lambda ph, j: (0,) * len(shape))


_mlp = pl.pallas_call(
    _mlp_body,
    grid=(4, NBLK),
    in_specs=[
        _big_spec(NUM), _big_spec(EW),
        _w_spec((1, NUM)), _w_spec((1, NUM)),
        _w_spec((1, EW)), _w_spec((1, EW)),
        _w_spec((1, D)), _w_spec((1, D)),
        _w_spec((NUM, H1)), _w_spec((EW, H1)), _w_spec((D, H1)),
        _w_spec((1, H1)), _w_spec((1, H1)), _w_spec((1, H1)),
        _w_spec((H1, H2)), _w_spec((1, H2)), _w_spec((1, H2)), _w_spec((1, H2)),
        _w_spec((H2, 1)), _w_spec((1, 1)),
    ],
    out_specs=pl.BlockSpec((BLK, 1), lambda ph, j: (j, 0)),
    out_shape=jax.ShapeDtypeStruct((B, 1), jnp.float32),
    scratch_shapes=[
        pltpu.VMEM((B, H1), jnp.float32),
        pltpu.VMEM((B, H2), jnp.float32),
        pltpu.VMEM((1, NUM), jnp.float32), pltpu.VMEM((1, NUM), jnp.float32),
        pltpu.VMEM((1, EW), jnp.float32), pltpu.VMEM((1, EW), jnp.float32),
        pltpu.VMEM((1, D), jnp.float32), pltpu.VMEM((1, D), jnp.float32),
        pltpu.VMEM((1, H1), jnp.float32), pltpu.VMEM((1, H1), jnp.float32),
        pltpu.VMEM((1, H2), jnp.float32), pltpu.VMEM((1, H2), jnp.float32),
    ],
)


def _pad_fd(a):
    a = a.reshape(a.shape[:-1] + (NF, D))
    pad = [(0, 0)] * (a.ndim - 1) + [(0, DP - D)]
    return jnp.pad(a, pad).reshape(a.shape[:-2] + (NF * DP,))


def kernel(numerical, categorical, tables, bn0_g, bn0_b, w1, b1, bn1_g,
           bn1_b, w2, b2, bn2_g, bn2_b, w3, b3):
    tab_pad = _repack(jnp.transpose(tables, (2, 0, 1))).reshape(NF * VP, DP)
    cat = categorical.astype(jnp.int32)
    offs = (jnp.arange(NF, dtype=jnp.int32) * VP)[None, :]
    flat_idx = (offs + (cat // VB) * VB + (cat % RR) * 8
                + (cat % VB) // RR).reshape(NW, NG, G)
    emb = _make_gather()(tab_pad, flat_idx).reshape(B, EW)

    e0, e1 = NUM, NUM + NF * D
    out = _mlp(
        numerical, emb,
        bn0_g[None, :e0], bn0_b[None, :e0],
        _pad_fd(bn0_g[e0:e1])[None, :], _pad_fd(bn0_b[e0:e1])[None, :],
        bn0_g[None, e1:], bn0_b[None, e1:],
        w1[:, :e0].T, _pad_fd(w1[:, e0:e1]).T, w1[:, e1:].T,
        b1[None, :], bn1_g[None, :], bn1_b[None, :],
        w2.T, b2[None, :], bn2_g[None, :], bn2_b[None, :],
        w3.T, b3[None, :],
    )
    return out


# speedup vs baseline: 5.5725x; 1.0309x over previous
import functools

import jax
import jax.numpy as jnp
from jax import lax
from jax.experimental import pallas as pl
from jax.experimental.pallas import tpu as pltpu
from jax.experimental.pallas import tpu_sc as plsc

B = 16384
NF = 26
V = 100000
D = 10
DP = 16
NUM = 13
H1 = 128
H2 = 64
EW = NF * DP

NC = 2
NS = 16
NW = NC * NS
ROWS = B * NF
RPW = ROWS // NW
G = 128
NG = RPW // G
CH = 26
NCH = NG // CH
CROWS = CH * G


def _gather_body(tab_hbm, idx_hbm, out_hbm, idx_v, buf0, buf1,
                 gsem0, gsem1, osem0, osem1):
    wid = lax.axis_index("s") * NC + lax.axis_index("c")
    pltpu.sync_copy(idx_hbm.at[wid], idx_v)
    bufs = (buf0, buf1)
    gsems = (gsem0, gsem1)
    osems = (osem0, osem1)
    cps = [None, None]
    ocs = [None, None]

    def fire(ch):
        b = ch & 1
        cps[b] = [
            pltpu.async_copy(
                tab_hbm.at[idx_v.at[ch * CH + j]],
                bufs[b].at[pl.ds(j * G, G)],
                gsems[b])
            for j in range(CH)
        ]

    fire(0)
    for ch in range(NCH):
        b = ch & 1
        if ch + 1 < NCH:
            nb = (ch + 1) & 1
            if ocs[nb] is not None:
                ocs[nb].wait()
            fire(ch + 1)
        for cp in cps[b]:
            cp.wait()
        oc = pltpu.make_async_copy(
            bufs[b], out_hbm.at[pl.ds(wid * RPW + ch * CROWS, CROWS)],
            osems[b])
        oc.start()
        ocs[b] = oc
    ocs[(NCH - 2) & 1].wait()
    ocs[(NCH - 1) & 1].wait()


@functools.lru_cache(maxsize=1)
def _make_gather():
    return pl.kernel(
        _gather_body,
        out_type=jax.ShapeDtypeStruct((ROWS, DP), jnp.float32),
        mesh=plsc.VectorSubcoreMesh(
            core_axis_name="c", subcore_axis_name="s",
            num_cores=NC, num_subcores=NS),
        scratch_types=[
            pltpu.VMEM((NG, G), jnp.int32),
            pltpu.VMEM((CROWS, DP), jnp.float32),
            pltpu.VMEM((CROWS, DP), jnp.float32),
            pltpu.SemaphoreType.DMA,
            pltpu.SemaphoreType.DMA,
            pltpu.SemaphoreType.DMA,
            pltpu.SemaphoreType.DMA,
        ],
        compiler_params=pltpu.CompilerParams(use_tc_tiling_on_sc=False),
    )


VB = 4096
NVB = -(-V // VB)
VP = NVB * VB
QF = VP // 8
RR = VB // 8


def _repack_body(tp_ref, out_ref):
    x = tp_ref[...]
    xc = jnp.concatenate(
        [x[:, :, RR * s:RR * (s + 1)] for s in range(8)], axis=0)
    i0 = lax.broadcasted_iota(jnp.int32, (8 * D, 128), 0)
    i1 = lax.broadcasted_iota(jnp.int32, (8 * D, 128), 1)
    place = (i1 == (i0 // D) * DP + (i0 % D)).astype(jnp.float32)
    out_ref[...] = lax.dot_general(
        xc, place, (((0,), (0,)), ((), ())), preferred_element_type=jnp.float32)


_repack = pl.pallas_call(
    _repack_body,
    grid=(NVB,),
    in_specs=[pl.BlockSpec((D, NF, VB), lambda v: (0, 0, v))],
    out_specs=pl.BlockSpec((NF, RR, 128), lambda v: (0, v, 0)),
    out_shape=jax.ShapeDtypeStruct((NF, QF, 128), jnp.float32),
)


BLK = 4096
NBLK = B // BLK
_EPS = 1e-5


def _fm_from_emb(emb):
    i0 = lax.broadcasted_iota(jnp.int32, (EW, D), 0)
    i1 = lax.broadcasted_iota(jnp.int32, (EW, D), 1)
    sel = (i0 % DP == i1).astype(jnp.float32)
    sum_emb = jnp.dot(emb, sel, preferred_element_type=jnp.float32)
    sq_sum = jnp.dot(emb * emb, sel, preferred_element_type=jnp.float32)
    return 0.5 * (sum_emb * sum_emb - sq_sum)


def _mlp_body(num_ref, emb_ref,
              g0n_ref, b0n_ref, g0e_ref, b0e_ref, g0f_ref, b0f_ref,
              w1n_ref, w1e_ref, w1f_ref, b1_ref, g1_ref, bb1_ref,
              w2_ref, b2_ref, g2_ref, bb2_ref, w3_ref, b3_ref,
              out_ref,
              h1_s, h2_s,
              sn_s, qn_s, se_s, qe_s, sf_s, qf_s, s1_s, q1_s, s2_s, q2_s):
    f32 = jnp.float32
    ph = pl.program_id(0)
    j = pl.program_id(1)
    rows = pl.ds(j * BLK, BLK)

    def colstats(p):
        return (jnp.sum(p, axis=0, keepdims=True),
                jnp.sum(p * p, axis=0, keepdims=True))

    def scale_shift(s_ref, q_ref, g, b):
        m = s_ref[...] * (1.0 / B)
        v = q_ref[...] * (1.0 / B) - m * m
        sc = g / jnp.sqrt(v + _EPS)
        return sc, b - m * sc

    @pl.when(ph == 0)
    def _phase0():
        @pl.when(j == 0)
        def _zero():
            for r in (sn_s, qn_s, se_s, qe_s, sf_s, qf_s, s1_s, q1_s, s2_s, q2_s):
                r[...] = jnp.zeros_like(r)
        num = num_ref[...]
        emb = emb_ref[...]
        fm = _fm_from_emb(emb)
        s, q = colstats(num); sn_s[...] += s; qn_s[...] += q
        s, q = colstats(emb); se_s[...] += s; qe_s[...] += q
        s, q = colstats(fm); sf_s[...] += s; qf_s[...] += q

    @pl.when(ph == 1)
    def _phase1():
        num = num_ref[...]
        emb = emb_ref[...]
        fm = _fm_from_emb(emb)
        scn, shn = scale_shift(sn_s, qn_s, g0n_ref[...], b0n_ref[...])
        sce, she = scale_shift(se_s, qe_s, g0e_ref[...], b0e_ref[...])
        scf, shf = scale_shift(sf_s, qf_s, g0f_ref[...], b0f_ref[...])
        h1 = (jnp.dot(num * scn + shn, w1n_ref[...], preferred_element_type=f32)
              + jnp.dot(emb * sce + she, w1e_ref[...], preferred_element_type=f32)
              + jnp.dot(fm * scf + shf, w1f_ref[...], preferred_element_type=f32)
              + b1_ref[...])
        s, q = colstats(h1); s1_s[...] += s; q1_s[...] += q
        h1_s[rows, :] = h1

    @pl.when(ph == 2)
    def _phase2():
        h1 = h1_s[rows, :]
        sc, sh = scale_shift(s1_s, q1_s, g1_ref[...], bb1_ref[...])
        h1 = jnp.maximum(h1 * sc + sh, 0.0)
        h2 = jnp.dot(h1, w2_ref[...], preferred_element_type=f32) + b2_ref[...]
        s, q = colstats(h2); s2_s[...] += s; q2_s[...] += q
        h2_s[rows, :] = h2

    @pl.when(ph == 3)
    def _phase3():
        h2 = h2_s[rows, :]
        sc, sh = scale_shift(s2_s, q2_s, g2_ref[...], bb2_ref[...])
        h2 = jnp.maximum(h2 * sc + sh, 0.0)
        out_ref[...] = jnp.dot(h2, w3_ref[...], preferred_element_type=f32) + b3_ref[...]


def _big_spec(ncols):
    return pl.BlockSpec(
        (BLK, ncols), lambda ph, j: (jnp.where(ph <= 1, j, 0), 0))


def _w_spec(shape):
    return pl.BlockSpec(shape, lambda ph, j: (0,) * len(shape))


_mlp = pl.pallas_call(
    _mlp_body,
    grid=(4, NBLK),
    in_specs=[
        _big_spec(NUM), _big_spec(EW),
        _w_spec((1, NUM)), _w_spec((1, NUM)),
        _w_spec((1, EW)), _w_spec((1, EW)),
        _w_spec((1, D)), _w_spec((1, D)),
        _w_spec((NUM, H1)), _w_spec((EW, H1)), _w_spec((D, H1)),
        _w_spec((1, H1)), _w_spec((1, H1)), _w_spec((1, H1)),
        _w_spec((H1, H2)), _w_spec((1, H2)), _w_spec((1, H2)), _w_spec((1, H2)),
        _w_spec((H2, 1)), _w_spec((1, 1)),
    ],
    out_specs=pl.BlockSpec((BLK, 1), lambda ph, j: (j, 0)),
    out_shape=jax.ShapeDtypeStruct((B, 1), jnp.float32),
    scratch_shapes=[
        pltpu.VMEM((B, H1), jnp.float32),
        pltpu.VMEM((B, H2), jnp.float32),
        pltpu.VMEM((1, NUM), jnp.float32), pltpu.VMEM((1, NUM), jnp.float32),
        pltpu.VMEM((1, EW), jnp.float32), pltpu.VMEM((1, EW), jnp.float32),
        pltpu.VMEM((1, D), jnp.float32), pltpu.VMEM((1, D), jnp.float32),
        pltpu.VMEM((1, H1), jnp.float32), pltpu.VMEM((1, H1), jnp.float32),
        pltpu.VMEM((1, H2), jnp.float32), pltpu.VMEM((1, H2), jnp.float32),
    ],
)


def _pad_fd(a):
    a = a.reshape(a.shape[:-1] + (NF, D))
    pad = [(0, 0)] * (a.ndim - 1) + [(0, DP - D)]
    return jnp.pad(a, pad).reshape(a.shape[:-2] + (NF * DP,))


def kernel(numerical, categorical, tables, bn0_g, bn0_b, w1, b1, bn1_g,
           bn1_b, w2, b2, bn2_g, bn2_b, w3, b3):
    tab_pad = _repack(jnp.transpose(tables, (2, 0, 1))).reshape(NF * VP, DP)
    cat = categorical.astype(jnp.int32)
    offs = (jnp.arange(NF, dtype=jnp.int32) * VP)[None, :]
    flat_idx = (offs + (cat // VB) * VB + (cat % RR) * 8
                + (cat % VB) // RR).reshape(NW, NG, G)
    emb = _make_gather()(tab_pad, flat_idx).reshape(B, EW)

    e0, e1 = NUM, NUM + NF * D
    out = _mlp(
        numerical, emb,
        bn0_g[None, :e0], bn0_b[None, :e0],
        _pad_fd(bn0_g[e0:e1])[None, :], _pad_fd(bn0_b[e0:e1])[None, :],
        bn0_g[None, e1:], bn0_b[None, e1:],
        w1[:, :e0].T, _pad_fd(w1[:, e0:e1]).T, w1[:, e1:].T,
        b1[None, :], bn1_g[None, :], bn1_b[None, :],
        w2.T, b2[None, :], bn2_g[None, :], bn2_b[None, :],
        w3.T, b3[None, :],
    )
    return out
